# Initial kernel scaffold; baseline (speedup 1.0000x reference)
import functools

import jax
import jax.numpy as jnp
from jax import lax
from jax.experimental import pallas as pl
from jax.experimental.pallas import tpu as pltpu
from jax.experimental.pallas import tpu_sc as plsc

N = 10000
NPAD = 10240
D = 128
E = 320000
CH = 128
TPB = 79
EPAD = 32 * TPB * CH
DUMMY = 10100
RPT = NPAD // 16
ALPHA = 0.1
NUM_LAYERS = 4
BN = 1024


def _mesh():
    return plsc.VectorSubcoreMesh(
        core_axis_name="c", subcore_axis_name="s", num_cores=2, num_subcores=16)


def _make_propagate(W):
    def body(g_hbm, src_hbm, dst_hbm, zz_hbm, out_hbm, srcb, dstb, rows, acc, semg):
        c = lax.axis_index("c")
        s = lax.axis_index("s")
        w = s * 2 + c
        pltpu.sync_copy(src_hbm.at[pl.ds(w * TPB, TPB)], srcb)
        pltpu.sync_copy(dst_hbm.at[pl.ds(w * TPB, TPB)], dstb)
        pltpu.sync_copy(zz_hbm, rows)
        for k in range(RPT // CH):
            pltpu.sync_copy(rows, acc.at[pl.ds(s * RPT + k * CH, CH)])
        plsc.subcore_barrier()

        def step(j, carry):
            pltpu.async_copy(g_hbm.at[srcb.at[j]], rows, semg).wait()
            pltpu.sync_copy(rows, acc.at[dstb.at[j]], add=True)
            return carry

        lax.fori_loop(0, TPB, step, 0)
        plsc.subcore_barrier()
        for k in range(RPT // CH):
            sl = pl.ds(s * RPT + k * CH, CH)
            pltpu.sync_copy(acc.at[sl], rows)
            pltpu.sync_copy(rows, out_hbm.at[c].at[sl])

    return pl.kernel(
        body,
        out_type=jax.ShapeDtypeStruct((2, NPAD, W), jnp.float32),
        mesh=_mesh(),
        scratch_types=[
            pltpu.VMEM((TPB, CH), jnp.int32),
            pltpu.VMEM((TPB, CH), jnp.int32),
            pltpu.VMEM((CH, W), jnp.float32),
            pltpu.VMEM_SHARED((NPAD, W), jnp.float32),
            pltpu.SemaphoreType.DMA,
        ],
    )


def _make_degree():
    def body(dst_hbm, ones_hbm, zz_hbm, out_hbm, dstb, rows, acc):
        c = lax.axis_index("c")
        s = lax.axis_index("s")
        w = s * 2 + c
        pltpu.sync_copy(dst_hbm.at[pl.ds(w * TPB, TPB)], dstb)
        pltpu.sync_copy(zz_hbm, rows)
        for k in range(RPT // CH):
            pltpu.sync_copy(rows, acc.at[pl.ds(s * RPT + k * CH, CH)])
        plsc.subcore_barrier()
        pltpu.sync_copy(ones_hbm, rows)

        def step(j, carry):
            pltpu.sync_copy(rows, acc.at[dstb.at[j]], add=True)
            return carry

        lax.fori_loop(0, TPB, step, 0)
        plsc.subcore_barrier()
        for k in range(RPT // CH):
            sl = pl.ds(s * RPT + k * CH, CH)
            pltpu.sync_copy(acc.at[sl], rows)
            pltpu.sync_copy(rows, out_hbm.at[c].at[sl])

    return pl.kernel(
        body,
        out_type=jax.ShapeDtypeStruct((2, NPAD, 16), jnp.float32),
        mesh=_mesh(),
        scratch_types=[
            pltpu.VMEM((TPB, CH), jnp.int32),
            pltpu.VMEM((CH, 16), jnp.float32),
            pltpu.VMEM_SHARED((NPAD, 16), jnp.float32),
        ],
    )


_GRID = (NPAD // BN,)


def _row_spec(w):
    return pl.BlockSpec((BN, w), lambda i: (i, 0))


def _pair_spec(w):
    return pl.BlockSpec((2, BN, w), lambda i: (0, i, 0))


def _full_spec(a, b):
    return pl.BlockSpec((a, b), lambda i: (0, 0))


def _prep_body(x_ref, degp_ref, dinv128_ref, g0_ref, ax0_ref, dinv16_ref):
    deg = 1.0 + degp_ref[0, :, 0:1] + degp_ref[1, :, 0:1]
    dinv = lax.rsqrt(deg)
    dinv128_ref[...] = jnp.broadcast_to(dinv, (BN, D))
    g0_ref[...] = x_ref[...] * dinv
    ax0_ref[...] = ALPHA * x_ref[...]
    dinv16_ref[...] = jnp.broadcast_to(dinv, (BN, 16))


_prep = pl.pallas_call(
    _prep_body,
    grid=_GRID,
    in_specs=[_row_spec(D), _pair_spec(16)],
    out_specs=[_row_spec(D), _row_spec(D), _row_spec(D), _row_spec(16)],
    out_shape=[
        jax.ShapeDtypeStruct((NPAD, D), jnp.float32),
        jax.ShapeDtypeStruct((NPAD, D), jnp.float32),
        jax.ShapeDtypeStruct((NPAD, D), jnp.float32),
        jax.ShapeDtypeStruct((NPAD, 16), jnp.float32),
    ],
)


def _layer_body(sp_ref, g_ref, ax0_ref, dinv_ref, w_ref, out_ref):
    dinv = dinv_ref[...]
    u = (1.0 - ALPHA) * dinv * (sp_ref[0] + sp_ref[1] + g_ref[...]) + ax0_ref[...]
    h = lax.dot_general(u, w_ref[...], (((1,), (0,)), ((), ())),
                        preferred_element_type=jnp.float32)
    out_ref[...] = dinv * jnp.maximum(h, 0.0)


_layer = pl.pallas_call(
    _layer_body,
    grid=_GRID,
    in_specs=[_pair_spec(D), _row_spec(D), _row_spec(D), _row_spec(D), _full_spec(D, D)],
    out_specs=_row_spec(D),
    out_shape=jax.ShapeDtypeStruct((NPAD, D), jnp.float32),
)


def _layer4_body(sp_ref, g_ref, ax0_ref, dinv_ref, w_ref, wa_ref, out_ref):
    dinv = dinv_ref[...]
    u = (1.0 - ALPHA) * dinv * (sp_ref[0] + sp_ref[1] + g_ref[...]) + ax0_ref[...]
    h = lax.dot_general(u, w_ref[...], (((1,), (0,)), ((), ())),
                        preferred_element_type=jnp.float32)
    g4 = dinv * jnp.maximum(h, 0.0)
    out_ref[...] = lax.dot_general(g4, wa_ref[...], (((1,), (0,)), ((), ())),
                                   preferred_element_type=jnp.float32)


_layer4 = pl.pallas_call(
    _layer4_body,
    grid=_GRID,
    in_specs=[_pair_spec(D), _row_spec(D), _row_spec(D), _row_spec(D),
              _full_spec(D, D), _full_spec(D, 16)],
    out_specs=_row_spec(16),
    out_shape=jax.ShapeDtypeStruct((NPAD, 16), jnp.float32),
)


def _mid_body(syp_ref, y_ref, dinv16_ref, ba_ref, out_ref):
    d = dinv16_ref[...]
    out_ref[...] = d * d * (syp_ref[0] + syp_ref[1] + y_ref[...]) + d * ba_ref[...]


_mid = pl.pallas_call(
    _mid_body,
    grid=_GRID,
    in_specs=[_pair_spec(16), _row_spec(16), _row_spec(16), _full_spec(1, 16)],
    out_specs=_row_spec(16),
    out_shape=jax.ShapeDtypeStruct((NPAD, 16), jnp.float32),
)


def _fin_body(sup_ref, u2_ref, dinv16_ref, wb_ref, bb_ref, out_ref):
    q = dinv16_ref[...] * (sup_ref[0] + sup_ref[1] + u2_ref[...])
    hb = jnp.sum(q * wb_ref[...], axis=1, keepdims=True) + bb_ref[...]
    out_ref[...] = jax.nn.sigmoid(hb)


_fin = pl.pallas_call(
    _fin_body,
    grid=_GRID,
    in_specs=[_pair_spec(16), _row_spec(16), _row_spec(16), _full_spec(1, 16),
              _full_spec(1, 1)],
    out_specs=_row_spec(1),
    out_shape=jax.ShapeDtypeStruct((NPAD, 1), jnp.float32),
)


def kernel(x, edge_index, W_gcn2, W_a, b_a, W_b, b_b):
    x_pad = jnp.pad(x, ((0, NPAD - N), (0, 0)))
    pad = jnp.full((EPAD - E,), DUMMY, dtype=jnp.int32)
    src2d = jnp.concatenate([edge_index[0], pad]).reshape(32 * TPB, CH)
    dst2d = jnp.concatenate([edge_index[1], pad]).reshape(32 * TPB, CH)
    z128 = jnp.zeros((CH, D), jnp.float32)
    z16 = jnp.zeros((CH, 16), jnp.float32)
    ones16 = jnp.ones((CH, 16), jnp.float32)

    prop128 = _make_propagate(D)
    prop16 = _make_propagate(16)
    degree = _make_degree()

    degp = degree(dst2d, ones16, z16)
    dinv128, g, ax0, dinv16 = _prep(x_pad, degp)
    for l in range(NUM_LAYERS):
        sp = prop128(g, src2d, dst2d, z128)
        if l < NUM_LAYERS - 1:
            g = _layer(sp, g, ax0, dinv128, W_gcn2[l])
        else:
            y = _layer4(sp, g, ax0, dinv128, W_gcn2[l], W_a)
    syp = prop16(y, src2d, dst2d, z16)
    u2 = _mid(syp, y, dinv16, b_a.reshape(1, 16))
    sup = prop16(u2, src2d, dst2d, z16)
    res = _fin(sup, u2, dinv16, W_b.reshape(1, 16), b_b.reshape(1, 1))
    return res[:N, 0]


# speedup vs baseline: 5.0293x; 5.0293x over previous
import functools

import jax
import jax.numpy as jnp
from jax import lax
from jax.experimental import pallas as pl
from jax.experimental.pallas import tpu as pltpu
from jax.experimental.pallas import tpu_sc as plsc

N = 10000
NPAD = 10240
D = 128
E = 320000
CH = 128
TPB = 80
EPAD = 32 * TPB * CH
DUMMY = 10100
RPT = NPAD // 16
ALPHA = 0.1
NUM_LAYERS = 4
BN = 1024


def _mesh():
    return plsc.VectorSubcoreMesh(
        core_axis_name="c", subcore_axis_name="s", num_cores=2, num_subcores=16)


def _make_propagate(W):
    def body(g_hbm, src_hbm, dst_hbm, zz_hbm, out_hbm, srcb, dstb, rows, acc, semg):
        c = lax.axis_index("c")
        s = lax.axis_index("s")
        w = s * 2 + c
        pltpu.sync_copy(src_hbm.at[pl.ds(w * TPB, TPB)], srcb)
        pltpu.sync_copy(dst_hbm.at[pl.ds(w * TPB, TPB)], dstb)
        pltpu.sync_copy(zz_hbm, rows)
        for k in range(RPT // CH):
            pltpu.sync_copy(rows, acc.at[pl.ds(s * RPT + k * CH, CH)])
        plsc.subcore_barrier()

        def step(j, carry):
            pltpu.async_copy(g_hbm.at[srcb.at[j]], rows, semg).wait()
            pltpu.sync_copy(rows, acc.at[dstb.at[j]], add=True)
            return carry

        lax.fori_loop(0, TPB, step, 0)
        plsc.subcore_barrier()
        for k in range(RPT // CH):
            sl = pl.ds(s * RPT + k * CH, CH)
            pltpu.sync_copy(acc.at[sl], rows)
            pltpu.sync_copy(rows, out_hbm.at[c].at[sl])

    return pl.kernel(
        body,
        out_type=jax.ShapeDtypeStruct((2, NPAD, W), jnp.float32),
        mesh=_mesh(),
        scratch_types=[
            pltpu.VMEM((TPB, CH), jnp.int32),
            pltpu.VMEM((TPB, CH), jnp.int32),
            pltpu.VMEM((CH, W), jnp.float32),
            pltpu.VMEM_SHARED((NPAD, W), jnp.float32),
            pltpu.SemaphoreType.DMA,
        ],
    )


def _make_degree():
    def body(dst_hbm, ones_hbm, zz_hbm, out_hbm, dstb, rows, acc):
        c = lax.axis_index("c")
        s = lax.axis_index("s")
        w = s * 2 + c
        pltpu.sync_copy(dst_hbm.at[pl.ds(w * TPB, TPB)], dstb)
        pltpu.sync_copy(zz_hbm, rows)
        for k in range(RPT // CH):
            pltpu.sync_copy(rows, acc.at[pl.ds(s * RPT + k * CH, CH)])
        plsc.subcore_barrier()
        pltpu.sync_copy(ones_hbm, rows)

        def step(j, carry):
            pltpu.sync_copy(rows, acc.at[dstb.at[j]], add=True)
            return carry

        lax.fori_loop(0, TPB, step, 0)
        plsc.subcore_barrier()
        for k in range(RPT // CH):
            sl = pl.ds(s * RPT + k * CH, CH)
            pltpu.sync_copy(acc.at[sl], rows)
            pltpu.sync_copy(rows, out_hbm.at[c].at[sl])

    return pl.kernel(
        body,
        out_type=jax.ShapeDtypeStruct((2, NPAD, 16), jnp.float32),
        mesh=_mesh(),
        scratch_types=[
            pltpu.VMEM((TPB, CH), jnp.int32),
            pltpu.VMEM((CH, 16), jnp.float32),
            pltpu.VMEM_SHARED((NPAD, 16), jnp.float32),
        ],
    )


_GRID = (NPAD // BN,)


def _row_spec(w):
    return pl.BlockSpec((BN, w), lambda i: (i, 0))


def _pair_spec(w):
    return pl.BlockSpec((2, BN, w), lambda i: (0, i, 0))


def _full_spec(a, b):
    return pl.BlockSpec((a, b), lambda i: (0, 0))


def _prep_body(x_ref, degp_ref, dinv128_ref, g0_ref, ax0_ref):
    deg = 1.0 + degp_ref[0, :, 0:1] + degp_ref[1, :, 0:1]
    dinv = lax.rsqrt(deg)
    dinv128_ref[...] = jnp.broadcast_to(dinv, (BN, D))
    g0_ref[...] = x_ref[...] * dinv
    ax0_ref[...] = ALPHA * x_ref[...]


_prep = pl.pallas_call(
    _prep_body,
    grid=_GRID,
    in_specs=[_row_spec(D), _pair_spec(D)],
    out_specs=[_row_spec(D), _row_spec(D), _row_spec(D)],
    out_shape=[
        jax.ShapeDtypeStruct((NPAD, D), jnp.float32),
        jax.ShapeDtypeStruct((NPAD, D), jnp.float32),
        jax.ShapeDtypeStruct((NPAD, D), jnp.float32),
    ],
)


def _layer_body(sp_ref, g_ref, ax0_ref, dinv_ref, w_ref, out_ref):
    dinv = dinv_ref[...]
    u = (1.0 - ALPHA) * dinv * (sp_ref[0] + sp_ref[1] + g_ref[...]) + ax0_ref[...]
    h = lax.dot_general(u, w_ref[...], (((1,), (0,)), ((), ())),
                        preferred_element_type=jnp.float32)
    out_ref[...] = dinv * jnp.maximum(h, 0.0)


_layer = pl.pallas_call(
    _layer_body,
    grid=_GRID,
    in_specs=[_pair_spec(D), _row_spec(D), _row_spec(D), _row_spec(D), _full_spec(D, D)],
    out_specs=_row_spec(D),
    out_shape=jax.ShapeDtypeStruct((NPAD, D), jnp.float32),
)


def _layer4_body(sp_ref, g_ref, ax0_ref, dinv_ref, w_ref, wa_ref, out_ref):
    dinv = dinv_ref[...]
    u = (1.0 - ALPHA) * dinv * (sp_ref[0] + sp_ref[1] + g_ref[...]) + ax0_ref[...]
    h = lax.dot_general(u, w_ref[...], (((1,), (0,)), ((), ())),
                        preferred_element_type=jnp.float32)
    g4 = dinv * jnp.maximum(h, 0.0)
    out_ref[...] = lax.dot_general(g4, wa_ref[...], (((1,), (0,)), ((), ())),
                                   preferred_element_type=jnp.float32)


_layer4 = pl.pallas_call(
    _layer4_body,
    grid=_GRID,
    in_specs=[_pair_spec(D), _row_spec(D), _row_spec(D), _row_spec(D),
              _full_spec(D, D), _full_spec(D, D)],
    out_specs=_row_spec(D),
    out_shape=jax.ShapeDtypeStruct((NPAD, D), jnp.float32),
)


def _mid_body(syp_ref, y_ref, dinv_ref, ba_ref, out_ref):
    d = dinv_ref[...]
    out_ref[...] = d * d * (syp_ref[0] + syp_ref[1] + y_ref[...]) + d * ba_ref[...]


_mid = pl.pallas_call(
    _mid_body,
    grid=_GRID,
    in_specs=[_pair_spec(D), _row_spec(D), _row_spec(D), _full_spec(1, D)],
    out_specs=_row_spec(D),
    out_shape=jax.ShapeDtypeStruct((NPAD, D), jnp.float32),
)


def _fin_body(sup_ref, u2_ref, dinv_ref, wb_ref, bb_ref, out_ref):
    q = dinv_ref[...] * (sup_ref[0] + sup_ref[1] + u2_ref[...])
    hb = jnp.sum(q * wb_ref[...], axis=1, keepdims=True) + bb_ref[...]
    out_ref[...] = jax.nn.sigmoid(hb)


_fin = pl.pallas_call(
    _fin_body,
    grid=_GRID,
    in_specs=[_pair_spec(D), _row_spec(D), _row_spec(D), _full_spec(1, D),
              _full_spec(1, 1)],
    out_specs=_row_spec(1),
    out_shape=jax.ShapeDtypeStruct((NPAD, 1), jnp.float32),
)


def kernel(x, edge_index, W_gcn2, W_a, b_a, W_b, b_b):
    x_pad = jnp.pad(x, ((0, NPAD - N), (0, 0)))
    pad = jnp.full((EPAD - E,), DUMMY, dtype=jnp.int32)
    src2d = jnp.concatenate([edge_index[0], pad]).reshape(32 * TPB, CH)
    dst2d = jnp.concatenate([edge_index[1], pad]).reshape(32 * TPB, CH)
    z128 = jnp.zeros((CH, D), jnp.float32)
    onesg = jnp.ones((NPAD, D), jnp.float32)

    wa128 = jnp.pad(W_a, ((0, 0), (0, D - 16)))
    ba128 = jnp.pad(b_a.reshape(1, 16), ((0, 0), (0, D - 16)))
    wb128 = jnp.pad(W_b.reshape(1, 16), ((0, 0), (0, D - 16)))

    prop128 = _make_propagate(D)

    degp = prop128(onesg, src2d, dst2d, z128)
    dinv128, g, ax0 = _prep(x_pad, degp)
    for l in range(NUM_LAYERS):
        sp = prop128(g, src2d, dst2d, z128)
        if l < NUM_LAYERS - 1:
            g = _layer(sp, g, ax0, dinv128, W_gcn2[l])
        else:
            y = _layer4(sp, g, ax0, dinv128, W_gcn2[l], wa128)
    syp = prop128(y, src2d, dst2d, z128)
    u2 = _mid(syp, y, dinv128, ba128)
    sup = prop128(u2, src2d, dst2d, z128)
    res = _fin(sup, u2, dinv128, wb128, b_b.reshape(1, 1))
    return res[:N, 0]


# speedup vs baseline: 5.5507x; 1.1037x over previous
import functools

import jax
import jax.numpy as jnp
from jax import lax
from jax.experimental import pallas as pl
from jax.experimental.pallas import tpu as pltpu
from jax.experimental.pallas import tpu_sc as plsc

N = 10000
NPAD = 10240
D = 128
E = 320000
CH = 128
TPB = 80
PH = 2
HT = TPB // PH
EPAD = 32 * TPB * CH
DUMMY = 10100
RPT = NPAD // 16
ALPHA = 0.1
NUM_LAYERS = 4
BN = 1024


def _mesh():
    return plsc.VectorSubcoreMesh(
        core_axis_name="c", subcore_axis_name="s", num_cores=2, num_subcores=16)


def _make_propagate(W):
    def body(g_hbm, src_hbm, dst_hbm, zz_hbm, out_hbm, srcb, dstb,
             rows0, rows1, acc, semg0, semg1, sems0, sems1):
        c = lax.axis_index("c")
        s = lax.axis_index("s")
        w = s * 2 + c
        pltpu.sync_copy(zz_hbm, rows0)
        for k in range(RPT // CH):
            pltpu.sync_copy(rows0, acc.at[pl.ds(s * RPT + k * CH, CH)])
        plsc.subcore_barrier()

        def gather(j, rows, sem):
            pltpu.async_copy(g_hbm.at[srcb.at[j]], rows, sem)

        def gwait(rows, sem):
            pltpu.make_async_copy(g_hbm.at[srcb.at[0]], rows, sem).wait()

        def scat(j, rows, sem):
            pltpu.async_copy(rows, acc.at[dstb.at[j]], sem, add=True)

        def swait(rows, sem):
            pltpu.make_async_copy(rows, acc.at[dstb.at[0]], sem).wait()

        def pair(k, carry):
            a = 2 * k
            gather(a + 1, rows1, semg1)
            gwait(rows0, semg0)
            scat(a, rows0, sems0)
            gwait(rows1, semg1)
            scat(a + 1, rows1, sems1)
            swait(rows0, sems0)
            gather(a + 2, rows0, semg0)
            swait(rows1, sems1)
            return carry

        for ph in range(PH):
            pltpu.sync_copy(src_hbm.at[pl.ds(w * TPB + ph * HT, HT)], srcb)
            pltpu.sync_copy(dst_hbm.at[pl.ds(w * TPB + ph * HT, HT)], dstb)
            gather(0, rows0, semg0)
            lax.fori_loop(0, HT // 2 - 1, pair, 0)
            gather(HT - 1, rows1, semg1)
            gwait(rows0, semg0)
            scat(HT - 2, rows0, sems0)
            gwait(rows1, semg1)
            scat(HT - 1, rows1, sems1)
            swait(rows0, sems0)
            swait(rows1, sems1)
        plsc.subcore_barrier()
        for k in range(RPT // CH):
            sl = pl.ds(s * RPT + k * CH, CH)
            pltpu.sync_copy(acc.at[sl], rows0)
            pltpu.sync_copy(rows0, out_hbm.at[c].at[sl])

    return pl.kernel(
        body,
        out_type=jax.ShapeDtypeStruct((2, NPAD, W), jnp.float32),
        mesh=_mesh(),
        scratch_types=[
            pltpu.VMEM((HT, CH), jnp.int32),
            pltpu.VMEM((HT, CH), jnp.int32),
            pltpu.VMEM((CH, W), jnp.float32),
            pltpu.VMEM((CH, W), jnp.float32),
            pltpu.VMEM_SHARED((NPAD, W), jnp.float32),
            pltpu.SemaphoreType.DMA,
            pltpu.SemaphoreType.DMA,
            pltpu.SemaphoreType.DMA,
            pltpu.SemaphoreType.DMA,
        ],
    )


def _make_degree():
    def body(dst_hbm, ones_hbm, zz_hbm, out_hbm, dstb, rows, acc):
        c = lax.axis_index("c")
        s = lax.axis_index("s")
        w = s * 2 + c
        pltpu.sync_copy(dst_hbm.at[pl.ds(w * TPB, TPB)], dstb)
        pltpu.sync_copy(zz_hbm, rows)
        for k in range(RPT // CH):
            pltpu.sync_copy(rows, acc.at[pl.ds(s * RPT + k * CH, CH)])
        plsc.subcore_barrier()
        pltpu.sync_copy(ones_hbm, rows)

        def step(j, carry):
            pltpu.sync_copy(rows, acc.at[dstb.at[j]], add=True)
            return carry

        lax.fori_loop(0, TPB, step, 0)
        plsc.subcore_barrier()
        for k in range(RPT // CH):
            sl = pl.ds(s * RPT + k * CH, CH)
            pltpu.sync_copy(acc.at[sl], rows)
            pltpu.sync_copy(rows, out_hbm.at[c].at[sl])

    return pl.kernel(
        body,
        out_type=jax.ShapeDtypeStruct((2, NPAD, 16), jnp.float32),
        mesh=_mesh(),
        scratch_types=[
            pltpu.VMEM((TPB, CH), jnp.int32),
            pltpu.VMEM((CH, 16), jnp.float32),
            pltpu.VMEM_SHARED((NPAD, 16), jnp.float32),
        ],
    )


_GRID = (NPAD // BN,)


def _row_spec(w):
    return pl.BlockSpec((BN, w), lambda i: (i, 0))


def _pair_spec(w):
    return pl.BlockSpec((2, BN, w), lambda i: (0, i, 0))


def _full_spec(a, b):
    return pl.BlockSpec((a, b), lambda i: (0, 0))


def _prep_body(x_ref, degp_ref, dinv128_ref, g0_ref, ax0_ref):
    deg = 1.0 + degp_ref[0, :, 0:1] + degp_ref[1, :, 0:1]
    dinv = lax.rsqrt(deg)
    dinv128_ref[...] = jnp.broadcast_to(dinv, (BN, D))
    g0_ref[...] = x_ref[...] * dinv
    ax0_ref[...] = ALPHA * x_ref[...]


_prep = pl.pallas_call(
    _prep_body,
    grid=_GRID,
    in_specs=[_row_spec(D), _pair_spec(D)],
    out_specs=[_row_spec(D), _row_spec(D), _row_spec(D)],
    out_shape=[
        jax.ShapeDtypeStruct((NPAD, D), jnp.float32),
        jax.ShapeDtypeStruct((NPAD, D), jnp.float32),
        jax.ShapeDtypeStruct((NPAD, D), jnp.float32),
    ],
)


def _layer_body(sp_ref, g_ref, ax0_ref, dinv_ref, w_ref, out_ref):
    dinv = dinv_ref[...]
    u = (1.0 - ALPHA) * dinv * (sp_ref[0] + sp_ref[1] + g_ref[...]) + ax0_ref[...]
    h = lax.dot_general(u, w_ref[...], (((1,), (0,)), ((), ())),
                        preferred_element_type=jnp.float32)
    out_ref[...] = dinv * jnp.maximum(h, 0.0)


_layer = pl.pallas_call(
    _layer_body,
    grid=_GRID,
    in_specs=[_pair_spec(D), _row_spec(D), _row_spec(D), _row_spec(D), _full_spec(D, D)],
    out_specs=_row_spec(D),
    out_shape=jax.ShapeDtypeStruct((NPAD, D), jnp.float32),
)


def _layer4_body(sp_ref, g_ref, ax0_ref, dinv_ref, w_ref, wa_ref, out_ref):
    dinv = dinv_ref[...]
    u = (1.0 - ALPHA) * dinv * (sp_ref[0] + sp_ref[1] + g_ref[...]) + ax0_ref[...]
    h = lax.dot_general(u, w_ref[...], (((1,), (0,)), ((), ())),
                        preferred_element_type=jnp.float32)
    g4 = dinv * jnp.maximum(h, 0.0)
    out_ref[...] = lax.dot_general(g4, wa_ref[...], (((1,), (0,)), ((), ())),
                                   preferred_element_type=jnp.float32)


_layer4 = pl.pallas_call(
    _layer4_body,
    grid=_GRID,
    in_specs=[_pair_spec(D), _row_spec(D), _row_spec(D), _row_spec(D),
              _full_spec(D, D), _full_spec(D, D)],
    out_specs=_row_spec(D),
    out_shape=jax.ShapeDtypeStruct((NPAD, D), jnp.float32),
)


def _mid_body(syp_ref, y_ref, dinv_ref, ba_ref, out_ref):
    d = dinv_ref[...]
    out_ref[...] = d * d * (syp_ref[0] + syp_ref[1] + y_ref[...]) + d * ba_ref[...]


_mid = pl.pallas_call(
    _mid_body,
    grid=_GRID,
    in_specs=[_pair_spec(D), _row_spec(D), _row_spec(D), _full_spec(1, D)],
    out_specs=_row_spec(D),
    out_shape=jax.ShapeDtypeStruct((NPAD, D), jnp.float32),
)


def _fin_body(sup_ref, u2_ref, dinv_ref, wb_ref, bb_ref, out_ref):
    q = dinv_ref[...] * (sup_ref[0] + sup_ref[1] + u2_ref[...])
    hb = jnp.sum(q * wb_ref[...], axis=1, keepdims=True) + bb_ref[...]
    out_ref[...] = jax.nn.sigmoid(hb)


_fin = pl.pallas_call(
    _fin_body,
    grid=_GRID,
    in_specs=[_pair_spec(D), _row_spec(D), _row_spec(D), _full_spec(1, D),
              _full_spec(1, 1)],
    out_specs=_row_spec(1),
    out_shape=jax.ShapeDtypeStruct((NPAD, 1), jnp.float32),
)


def kernel(x, edge_index, W_gcn2, W_a, b_a, W_b, b_b):
    x_pad = jnp.pad(x, ((0, NPAD - N), (0, 0)))
    pad = jnp.full((EPAD - E,), DUMMY, dtype=jnp.int32)
    src2d = jnp.concatenate([edge_index[0], pad]).reshape(32 * TPB, CH)
    dst2d = jnp.concatenate([edge_index[1], pad]).reshape(32 * TPB, CH)
    z128 = jnp.zeros((CH, D), jnp.float32)
    onesg = jnp.ones((NPAD, D), jnp.float32)

    wa128 = jnp.pad(W_a, ((0, 0), (0, D - 16)))
    ba128 = jnp.pad(b_a.reshape(1, 16), ((0, 0), (0, D - 16)))
    wb128 = jnp.pad(W_b.reshape(1, 16), ((0, 0), (0, D - 16)))

    prop128 = _make_propagate(D)

    degp = prop128(onesg, src2d, dst2d, z128)
    dinv128, g, ax0 = _prep(x_pad, degp)
    for l in range(NUM_LAYERS):
        sp = prop128(g, src2d, dst2d, z128)
        if l < NUM_LAYERS - 1:
            g = _layer(sp, g, ax0, dinv128, W_gcn2[l])
        else:
            y = _layer4(sp, g, ax0, dinv128, W_gcn2[l], wa128)
    syp = prop128(y, src2d, dst2d, z128)
    u2 = _mid(syp, y, dinv128, ba128)
    sup = prop128(u2, src2d, dst2d, z128)
    res = _fin(sup, u2, dinv128, wb128, b_b.reshape(1, 1))
    return res[:N, 0]


# speedup vs baseline: 6.3724x; 1.1480x over previous
import functools

import jax
import jax.numpy as jnp
from jax import lax
from jax.experimental import pallas as pl
from jax.experimental.pallas import tpu as pltpu
from jax.experimental.pallas import tpu_sc as plsc

N = 10000
NPAD = 10240
D = 128
E = 320000
CH = 128
TPB = 80
PH = 2
HT = TPB // PH
EPAD = 32 * TPB * CH
DUMMY = 10100
RPT = NPAD // 16
ALPHA = 0.1
NUM_LAYERS = 4
BN = 1024


def _mesh():
    return plsc.VectorSubcoreMesh(
        core_axis_name="c", subcore_axis_name="s", num_cores=2, num_subcores=16)


def _make_propagate(W):
    def body(g_hbm, src_hbm, dst_hbm, zz_hbm, out_hbm, srcb, dstb,
             rows0, rows1, acc, semg0, semg1, sems0, sems1):
        c = lax.axis_index("c")
        s = lax.axis_index("s")
        w = s * 2 + c
        pltpu.sync_copy(zz_hbm, rows0)
        for k in range(RPT // CH):
            pltpu.sync_copy(rows0, acc.at[pl.ds(s * RPT + k * CH, CH)])
        plsc.subcore_barrier()

        def gather(j, rows, sem):
            pltpu.async_copy(g_hbm.at[srcb.at[j]], rows, sem)

        def gwait(rows, sem):
            pltpu.make_async_copy(g_hbm.at[srcb.at[0]], rows, sem).wait()

        def scat(j, rows, sem):
            pltpu.async_copy(rows, acc.at[dstb.at[j]], sem, add=True)

        def swait(rows, sem):
            pltpu.make_async_copy(rows, acc.at[dstb.at[0]], sem).wait()

        def pair(k, carry):
            a = 2 * k
            gather(a + 1, rows1, semg1)
            gwait(rows0, semg0)
            scat(a, rows0, sems0)
            gwait(rows1, semg1)
            scat(a + 1, rows1, sems1)
            swait(rows0, sems0)
            gather(a + 2, rows0, semg0)
            swait(rows1, sems1)
            return carry

        for ph in range(PH):
            pltpu.sync_copy(src_hbm.at[pl.ds(w * TPB + ph * HT, HT)], srcb)
            pltpu.sync_copy(dst_hbm.at[pl.ds(w * TPB + ph * HT, HT)], dstb)
            gather(0, rows0, semg0)
            lax.fori_loop(0, HT // 2 - 1, pair, 0)
            gather(HT - 1, rows1, semg1)
            gwait(rows0, semg0)
            scat(HT - 2, rows0, sems0)
            gwait(rows1, semg1)
            scat(HT - 1, rows1, sems1)
            swait(rows0, sems0)
            swait(rows1, sems1)
        plsc.subcore_barrier()
        for k in range(RPT // CH):
            sl = pl.ds(s * RPT + k * CH, CH)
            pltpu.sync_copy(acc.at[sl], rows0)
            pltpu.sync_copy(rows0, out_hbm.at[c].at[sl])

    return pl.kernel(
        body,
        out_type=jax.ShapeDtypeStruct((2, NPAD, W), jnp.float32),
        mesh=_mesh(),
        scratch_types=[
            pltpu.VMEM((HT, CH), jnp.int32),
            pltpu.VMEM((HT, CH), jnp.int32),
            pltpu.VMEM((CH, W), jnp.float32),
            pltpu.VMEM((CH, W), jnp.float32),
            pltpu.VMEM_SHARED((NPAD, W), jnp.float32),
            pltpu.SemaphoreType.DMA,
            pltpu.SemaphoreType.DMA,
            pltpu.SemaphoreType.DMA,
            pltpu.SemaphoreType.DMA,
        ],
    )


def _make_degree():
    def body(dst_hbm, ones_hbm, zz_hbm, out_hbm, dstb, rows, acc, sem0, sem1):
        c = lax.axis_index("c")
        s = lax.axis_index("s")
        w = s * 2 + c
        pltpu.sync_copy(dst_hbm.at[pl.ds(w * TPB, TPB)], dstb)
        pltpu.sync_copy(zz_hbm, rows)
        for k in range(RPT // CH):
            pltpu.sync_copy(rows, acc.at[pl.ds(s * RPT + k * CH, CH)])
        plsc.subcore_barrier()
        pltpu.sync_copy(ones_hbm, rows)

        def pair(k, carry):
            pltpu.async_copy(rows, acc.at[dstb.at[2 * k]], sem0, add=True)
            pltpu.async_copy(rows, acc.at[dstb.at[2 * k + 1]], sem1, add=True)
            pltpu.make_async_copy(rows, acc.at[dstb.at[0]], sem0).wait()
            pltpu.make_async_copy(rows, acc.at[dstb.at[0]], sem1).wait()
            return carry

        lax.fori_loop(0, TPB // 2, pair, 0)
        plsc.subcore_barrier()
        for k in range(RPT // CH):
            sl = pl.ds(s * RPT + k * CH, CH)
            pltpu.sync_copy(acc.at[sl], rows)
            pltpu.sync_copy(rows, out_hbm.at[c].at[sl])

    return pl.kernel(
        body,
        out_type=jax.ShapeDtypeStruct((2, NPAD, D), jnp.float32),
        mesh=_mesh(),
        scratch_types=[
            pltpu.VMEM((TPB, CH), jnp.int32),
            pltpu.VMEM((CH, D), jnp.float32),
            pltpu.VMEM_SHARED((NPAD, D), jnp.float32),
            pltpu.SemaphoreType.DMA,
            pltpu.SemaphoreType.DMA,
        ],
    )


_GRID = (NPAD // BN,)


def _row_spec(w):
    return pl.BlockSpec((BN, w), lambda i: (i, 0))


def _pair_spec(w):
    return pl.BlockSpec((2, BN, w), lambda i: (0, i, 0))


def _full_spec(a, b):
    return pl.BlockSpec((a, b), lambda i: (0, 0))


def _prep_body(x_ref, degp_ref, dinv128_ref, g0_ref, ax0_ref):
    deg = 1.0 + degp_ref[0, :, 0:1] + degp_ref[1, :, 0:1]
    dinv = lax.rsqrt(deg)
    dinv128_ref[...] = jnp.broadcast_to(dinv, (BN, D))
    g0_ref[...] = x_ref[...] * dinv
    ax0_ref[...] = ALPHA * x_ref[...]


_prep = pl.pallas_call(
    _prep_body,
    grid=_GRID,
    in_specs=[_row_spec(D), _pair_spec(D)],
    out_specs=[_row_spec(D), _row_spec(D), _row_spec(D)],
    out_shape=[
        jax.ShapeDtypeStruct((NPAD, D), jnp.float32),
        jax.ShapeDtypeStruct((NPAD, D), jnp.float32),
        jax.ShapeDtypeStruct((NPAD, D), jnp.float32),
    ],
)


def _layer_body(sp_ref, g_ref, ax0_ref, dinv_ref, w_ref, out_ref):
    dinv = dinv_ref[...]
    u = (1.0 - ALPHA) * dinv * (sp_ref[0] + sp_ref[1] + g_ref[...]) + ax0_ref[...]
    h = lax.dot_general(u, w_ref[...], (((1,), (0,)), ((), ())),
                        preferred_element_type=jnp.float32)
    out_ref[...] = dinv * jnp.maximum(h, 0.0)


_layer = pl.pallas_call(
    _layer_body,
    grid=_GRID,
    in_specs=[_pair_spec(D), _row_spec(D), _row_spec(D), _row_spec(D), _full_spec(D, D)],
    out_specs=_row_spec(D),
    out_shape=jax.ShapeDtypeStruct((NPAD, D), jnp.float32),
)


def _layer4_body(sp_ref, g_ref, ax0_ref, dinv_ref, w_ref, wa_ref, out_ref):
    dinv = dinv_ref[...]
    u = (1.0 - ALPHA) * dinv * (sp_ref[0] + sp_ref[1] + g_ref[...]) + ax0_ref[...]
    h = lax.dot_general(u, w_ref[...], (((1,), (0,)), ((), ())),
                        preferred_element_type=jnp.float32)
    g4 = dinv * jnp.maximum(h, 0.0)
    out_ref[...] = lax.dot_general(g4, wa_ref[...], (((1,), (0,)), ((), ())),
                                   preferred_element_type=jnp.float32)


_layer4 = pl.pallas_call(
    _layer4_body,
    grid=_GRID,
    in_specs=[_pair_spec(D), _row_spec(D), _row_spec(D), _row_spec(D),
              _full_spec(D, D), _full_spec(D, D)],
    out_specs=_row_spec(D),
    out_shape=jax.ShapeDtypeStruct((NPAD, D), jnp.float32),
)


def _mid_body(syp_ref, y_ref, dinv_ref, ba_ref, out_ref):
    d = dinv_ref[...]
    out_ref[...] = d * d * (syp_ref[0] + syp_ref[1] + y_ref[...]) + d * ba_ref[...]


_mid = pl.pallas_call(
    _mid_body,
    grid=_GRID,
    in_specs=[_pair_spec(D), _row_spec(D), _row_spec(D), _full_spec(1, D)],
    out_specs=_row_spec(D),
    out_shape=jax.ShapeDtypeStruct((NPAD, D), jnp.float32),
)


def _fin_body(sup_ref, u2_ref, dinv_ref, wb_ref, bb_ref, out_ref):
    q = dinv_ref[...] * (sup_ref[0] + sup_ref[1] + u2_ref[...])
    hb = jnp.sum(q * wb_ref[...], axis=1, keepdims=True) + bb_ref[...]
    out_ref[...] = jax.nn.sigmoid(hb)


_fin = pl.pallas_call(
    _fin_body,
    grid=_GRID,
    in_specs=[_pair_spec(D), _row_spec(D), _row_spec(D), _full_spec(1, D),
              _full_spec(1, 1)],
    out_specs=_row_spec(1),
    out_shape=jax.ShapeDtypeStruct((NPAD, 1), jnp.float32),
)


def kernel(x, edge_index, W_gcn2, W_a, b_a, W_b, b_b):
    x_pad = jnp.pad(x, ((0, NPAD - N), (0, 0)))
    pad = jnp.full((EPAD - E,), DUMMY, dtype=jnp.int32)
    src2d = jnp.concatenate([edge_index[0], pad]).reshape(32 * TPB, CH)
    dst2d = jnp.concatenate([edge_index[1], pad]).reshape(32 * TPB, CH)
    z128 = jnp.zeros((CH, D), jnp.float32)
    ones128 = jnp.ones((CH, D), jnp.float32)

    wa128 = jnp.pad(W_a, ((0, 0), (0, D - 16)))
    ba128 = jnp.pad(b_a.reshape(1, 16), ((0, 0), (0, D - 16)))
    wb128 = jnp.pad(W_b.reshape(1, 16), ((0, 0), (0, D - 16)))

    prop128 = _make_propagate(D)
    degree = _make_degree()

    degp = degree(dst2d, ones128, z128)
    dinv128, g, ax0 = _prep(x_pad, degp)
    for l in range(NUM_LAYERS):
        sp = prop128(g, src2d, dst2d, z128)
        if l < NUM_LAYERS - 1:
            g = _layer(sp, g, ax0, dinv128, W_gcn2[l])
        else:
            y = _layer4(sp, g, ax0, dinv128, W_gcn2[l], wa128)
    syp = prop128(y, src2d, dst2d, z128)
    u2 = _mid(syp, y, dinv128, ba128)
    sup = prop128(u2, src2d, dst2d, z128)
    res = _fin(sup, u2, dinv128, wb128, b_b.reshape(1, 1))
    return res[:N, 0]
